# baseline (speedup 1.0000x reference)
import functools

import jax
import jax.numpy as jnp
from jax import lax
from jax.experimental import pallas as pl
from jax.experimental.pallas import tpu as pltpu
from jax.experimental.pallas import tpu_sc as plsc

SEQ = 8192
EMB = 64
VOCAB = 1000000
NBUF = 8

_cached = None


def _build():
    global _cached
    if _cached is not None:
        return _cached

    info = plsc.get_sparse_core_info()
    nc, ns = info.num_cores, info.num_subcores
    nw = nc * ns
    bpw = SEQ // nw
    mesh = plsc.VectorSubcoreMesh(core_axis_name="c", subcore_axis_name="s")

    @functools.partial(
        pl.kernel,
        mesh=mesh,
        out_type=jax.ShapeDtypeStruct((EMB, SEQ), jnp.float32),
        scratch_types=[
            pltpu.VMEM((bpw + 16,), jnp.int32),
            pltpu.VMEM((EMB, bpw), jnp.float32),
            pltpu.VMEM((EMB, bpw), jnp.float32),
            [pltpu.VMEM((EMB, 128), jnp.float32) for _ in range(NBUF)],
            pltpu.SemaphoreType.DMA,
            pltpu.SemaphoreType.DMA,
        ],
        compiler_params=pltpu.CompilerParams(needs_layout_passes=False),
    )
    def k(x_hbm, tokt_hbm, post_hbm, outt_hbm,
          idx_v, pos_v, out_v, bufs, sem, gsem):
        wid = lax.axis_index("s") * nc + lax.axis_index("c")
        base = wid * bpw
        iota16 = lax.iota(jnp.int32, 16)
        pltpu.sync_copy(x_hbm.at[pl.ds(base, bpw)], idx_v.at[pl.ds(0, bpw)])
        idx_v[pl.ds(bpw, 16)] = jnp.zeros((16,), jnp.int32)
        pcp = pltpu.async_copy(
            post_hbm.at[:, pl.ds(SEQ - base - bpw, bpw)], pos_v, sem
        )

        def fire(col, b):
            c0 = pl.multiple_of((col >> 7) * 128, 128)
            pltpu.async_copy(tokt_hbm.at[:, pl.ds(c0, 128)], bufs[b], gsem)

        def wait_one(b):
            pltpu.make_async_copy(
                tokt_hbm.at[:, pl.ds(0, 128)], bufs[b], gsem
            ).wait()

        def extract(col, j_scalar, b):
            lane = jnp.broadcast_to(col & 127, (16,))
            j = jnp.broadcast_to(j_scalar, (16,))
            for p in range(EMB // 16):
                rows = iota16 + p * 16
                vals = plsc.load_gather(bufs[b], [rows, lane])
                plsc.store_scatter(out_v, [rows, j], vals)

        vec0 = idx_v[pl.ds(0, 16)]
        for t in range(NBUF):
            fire(vec0[t], t)

        def fetch_group(gg, carry):
            vec = idx_v[pl.ds(gg * 16, 16)]
            vecn = idx_v[pl.ds(gg * 16 + 16, 16)]
            for t in range(16):
                b = t % NBUF
                wait_one(b)
                extract(vec[t], gg * 16 + t, b)
                fire(vec[t + NBUF] if t < 16 - NBUF else vecn[t - 16 + NBUF], b)
            return carry

        lax.fori_loop(0, bpw // 16, fetch_group, 0)
        for b in range(NBUF):
            wait_one(b)
        pcp.wait()

        def body(d, carry):
            for g in range(bpw // 16):
                sl = pl.ds(g * 16, 16)
                rsl = pl.ds(bpw - (g + 1) * 16, 16)
                out_v[d, sl] = out_v[d, sl] + lax.rev(pos_v[d, rsl], (0,))
            return carry

        lax.fori_loop(0, EMB, body, 0)
        pltpu.sync_copy(out_v, outt_hbm.at[:, pl.ds(base, bpw)])

    _cached = k
    return _cached


def kernel(x, token_table, pos_table):
    outt = _build()(x.astype(jnp.int32), token_table.T, pos_table.T)
    return outt.T


# speedup vs baseline: 1.0613x; 1.0613x over previous
import functools

import jax
import jax.numpy as jnp
from jax import lax
from jax.experimental import pallas as pl
from jax.experimental.pallas import tpu as pltpu
from jax.experimental.pallas import tpu_sc as plsc

SEQ = 8192
EMB = 64
VOCAB = 1000000
NBUF = 8

_cached = None


def _build():
    global _cached
    if _cached is not None:
        return _cached

    info = plsc.get_sparse_core_info()
    nc, ns = info.num_cores, info.num_subcores
    nw = nc * ns
    bpw = SEQ // nw
    mesh = plsc.VectorSubcoreMesh(core_axis_name="c", subcore_axis_name="s")

    @functools.partial(
        pl.kernel,
        mesh=mesh,
        out_type=jax.ShapeDtypeStruct((EMB, SEQ), jnp.float32),
        scratch_types=[
            pltpu.VMEM((bpw,), jnp.int32),
            pltpu.VMEM((EMB, bpw), jnp.float32),
            pltpu.VMEM((EMB, bpw), jnp.float32),
            [pltpu.VMEM((EMB, 128), jnp.float32) for _ in range(NBUF)],
            pltpu.SemaphoreType.DMA,
            pltpu.SemaphoreType.DMA,
        ],
        compiler_params=pltpu.CompilerParams(needs_layout_passes=False),
    )
    def k(x_hbm, tokt_hbm, post_hbm, outt_hbm,
          idx_v, pos_v, out_v, bufs, sem, gsem):
        wid = lax.axis_index("s") * nc + lax.axis_index("c")
        base = wid * bpw
        iota16 = lax.iota(jnp.int32, 16)
        pltpu.sync_copy(x_hbm.at[pl.ds(base, bpw)], idx_v)
        pcp = pltpu.async_copy(
            post_hbm.at[:, pl.ds(SEQ - base - bpw, bpw)], pos_v, sem
        )

        def fire(vec, t, b):
            col = vec[t]
            c0 = pl.multiple_of((col >> 7) * 128, 128)
            return pltpu.async_copy(
                tokt_hbm.at[:, pl.ds(c0, 128)], bufs[b], gsem
            )

        def drain_extract(cp, vec, t, j0, b):
            col = vec[t]
            lane = jnp.broadcast_to(col & 127, (16,))
            j = jnp.broadcast_to(j0 + t, (16,))
            cp.wait()
            for p in range(EMB // 16):
                rows = iota16 + p * 16
                vals = plsc.load_gather(bufs[b], [rows, lane])
                plsc.store_scatter(out_v, [rows, j], vals)

        def fetch_group(gg, carry):
            vec = idx_v[pl.ds(gg * 16, 16)]
            j0 = gg * 16
            cps = [fire(vec, b, b) for b in range(NBUF)]
            for b in range(16 - NBUF):
                drain_extract(cps[b], vec, b, j0, b % NBUF)
                cps.append(fire(vec, NBUF + b, b % NBUF))
            for b in range(16 - NBUF, 16):
                drain_extract(cps[b], vec, b, j0, b % NBUF)
            return carry

        lax.fori_loop(0, bpw // 16, fetch_group, 0)
        pcp.wait()

        def body(d, carry):
            for g in range(bpw // 16):
                sl = pl.ds(g * 16, 16)
                rsl = pl.ds(bpw - (g + 1) * 16, 16)
                out_v[d, sl] = out_v[d, sl] + lax.rev(pos_v[d, rsl], (0,))
            return carry

        lax.fori_loop(0, EMB, body, 0)
        pltpu.sync_copy(out_v, outt_hbm.at[:, pl.ds(base, bpw)])

    _cached = k
    return _cached


def kernel(x, token_table, pos_table):
    outt = _build()(x.astype(jnp.int32), token_table.T, pos_table.T)
    return outt.T
